# baseline (speedup 1.0000x reference)
import jax
import jax.numpy as jnp
from jax import lax
from jax.experimental import pallas as pl

B, N = 256, 64
EMB = 64
D_INV = 256
ROWS = B * N

INV_BLK = 32
N1_BLK = 4

A_VOCAB = 128
C_VOCAB = 8
B_VOCAB = 8


def _onehot_t(idx_row, vocab):
    io = lax.broadcasted_iota(jnp.int32, (vocab, idx_row.shape[1]), 0)
    return (io == idx_row).astype(jnp.float32)


_TDOT = (((0,), (0,)), ((), ()))


def _invs_body(at_ref, ac_ref, ta_ref, tc_ref, w1a_ref, w1c_ref, b1_ref,
               w2_ref, b2_ref, out_ref):
    emb_a = []
    emb_c = []
    for i in range(INV_BLK):
        oh_a = _onehot_t(at_ref[i:i + 1, :], A_VOCAB)
        emb_a.append(lax.dot_general(oh_a, ta_ref[...], _TDOT,
                                     preferred_element_type=jnp.float32))
        oh_c = _onehot_t(ac_ref[i:i + 1, :], C_VOCAB)
        emb_c.append(lax.dot_general(oh_c, tc_ref[...], _TDOT,
                                     preferred_element_type=jnp.float32))
    ea = jnp.concatenate(emb_a, axis=0)
    ec = jnp.concatenate(emb_c, axis=0)
    h = (jnp.dot(ea, w1a_ref[...], preferred_element_type=jnp.float32)
         + jnp.dot(ec, w1c_ref[...], preferred_element_type=jnp.float32)
         + b1_ref[...])
    h = h * jax.nn.sigmoid(h)
    out = (jnp.dot(h, w2_ref[...], preferred_element_type=jnp.float32)
           + b2_ref[...])
    out_ref[...] = out.reshape(INV_BLK, 64, D_INV)


def _edges_body(btt_ref, tblt_ref, out_ref):
    tbl_t = tblt_ref[...]
    for n1 in range(N1_BLK):
        for n2 in range(N):
            oh = _onehot_t(btt_ref[n1, n2:n2 + 1, :], B_VOCAB)
            out_ref[n1, n2] = jnp.dot(tbl_t, oh,
                                      preferred_element_type=jnp.float32)


def kernel(atom_types, bond_types, atom_mask, atom_charges, atom_type_table,
           charge_table, bond_table, W1, b1, W2, b2):
    del atom_mask

    ta = jnp.pad(atom_type_table, ((0, A_VOCAB - atom_type_table.shape[0]), (0, 0)))
    tc = jnp.pad(charge_table, ((0, C_VOCAB - charge_table.shape[0]), (0, 0)))
    w1a, w1c = W1[:EMB], W1[EMB:]
    b1r = b1.reshape(1, D_INV)
    b2r = b2.reshape(1, D_INV)

    full = lambda shape: pl.BlockSpec(shape, lambda i: (0,) * len(shape))
    invs = pl.pallas_call(
        _invs_body,
        grid=(B // INV_BLK,),
        in_specs=[
            pl.BlockSpec((INV_BLK, 64), lambda i: (i, 0)),
            pl.BlockSpec((INV_BLK, 64), lambda i: (i, 0)),
            full((A_VOCAB, EMB)),
            full((C_VOCAB, EMB)),
            full((EMB, D_INV)),
            full((EMB, D_INV)),
            full((1, D_INV)),
            full((D_INV, D_INV)),
            full((1, D_INV)),
        ],
        out_specs=pl.BlockSpec((INV_BLK, 64, D_INV), lambda i: (i, 0, 0)),
        out_shape=jax.ShapeDtypeStruct((B, N, D_INV), jnp.float32),
    )(atom_types, atom_charges, ta, tc, w1a, w1c, b1r, W2, b2r)

    btt = jnp.transpose(bond_types, (1, 2, 0))
    tbl_t = jnp.transpose(bond_table, (1, 0))
    edges_t = pl.pallas_call(
        _edges_body,
        grid=(N // N1_BLK,),
        in_specs=[
            pl.BlockSpec((N1_BLK, N, B), lambda i: (i, 0, 0)),
            full((EMB, B_VOCAB)),
        ],
        out_specs=pl.BlockSpec((N1_BLK, N, EMB, B), lambda i: (i, 0, 0, 0)),
        out_shape=jax.ShapeDtypeStruct((N, N, EMB, B), jnp.float32),
    )(btt, tbl_t)

    edges = jnp.transpose(edges_t, (3, 0, 1, 2))
    return invs, edges


# speedup vs baseline: 1.1424x; 1.1424x over previous
import jax
import jax.numpy as jnp
from jax import lax
from jax.experimental import pallas as pl

B, N = 256, 64
EMB = 64
D_INV = 256

N1_BLK = 2
STEPS = N // N1_BLK
INV_BLK = B // STEPS

A_VOCAB = 128
C_VOCAB = 8
B_VOCAB = 8


def _onehot_t(idx_row, vocab):
    io = lax.broadcasted_iota(jnp.int32, (vocab, idx_row.shape[1]), 0)
    return (io == idx_row).astype(jnp.float32)


_TDOT = (((0,), (0,)), ((), ()))


def _body(btt_ref, at_ref, ac_ref, tblt_ref, ta_ref, tc_ref, w1a_ref,
          w1c_ref, b1_ref, w2_ref, b2_ref, edges_ref, invs_ref):
    tbl_t = tblt_ref[...]
    for n1 in range(N1_BLK):
        for n2 in range(N):
            oh = _onehot_t(btt_ref[n1, n2:n2 + 1, :], B_VOCAB)
            edges_ref[n1, n2] = jnp.dot(tbl_t, oh,
                                        preferred_element_type=jnp.float32)

    emb_a = []
    emb_c = []
    for i in range(INV_BLK):
        oh_a = _onehot_t(at_ref[i:i + 1, :], A_VOCAB)
        emb_a.append(lax.dot_general(oh_a, ta_ref[...], _TDOT,
                                     preferred_element_type=jnp.float32))
        oh_c = _onehot_t(ac_ref[i:i + 1, :], C_VOCAB)
        emb_c.append(lax.dot_general(oh_c, tc_ref[...], _TDOT,
                                     preferred_element_type=jnp.float32))
    ea = jnp.concatenate(emb_a, axis=0)
    ec = jnp.concatenate(emb_c, axis=0)
    h = (jnp.dot(ea, w1a_ref[...], preferred_element_type=jnp.float32)
         + jnp.dot(ec, w1c_ref[...], preferred_element_type=jnp.float32)
         + b1_ref[...])
    h = h * jax.nn.sigmoid(h)
    out = (jnp.dot(h, w2_ref[...], preferred_element_type=jnp.float32)
           + b2_ref[...])
    invs_ref[...] = out.reshape(INV_BLK, 64, D_INV)


def kernel(atom_types, bond_types, atom_mask, atom_charges, atom_type_table,
           charge_table, bond_table, W1, b1, W2, b2):
    del atom_mask

    ta = jnp.pad(atom_type_table, ((0, A_VOCAB - atom_type_table.shape[0]), (0, 0)))
    tc = jnp.pad(charge_table, ((0, C_VOCAB - charge_table.shape[0]), (0, 0)))
    w1a, w1c = W1[:EMB], W1[EMB:]
    b1r = b1.reshape(1, D_INV)
    b2r = b2.reshape(1, D_INV)

    btt = jnp.transpose(bond_types, (1, 2, 0))
    tbl_t = jnp.transpose(bond_table, (1, 0))

    full = lambda shape: pl.BlockSpec(shape, lambda i: (0,) * len(shape))
    edges_t, invs = pl.pallas_call(
        _body,
        grid=(STEPS,),
        in_specs=[
            pl.BlockSpec((N1_BLK, N, B), lambda i: (i, 0, 0)),
            pl.BlockSpec((INV_BLK, 64), lambda i: (i, 0)),
            pl.BlockSpec((INV_BLK, 64), lambda i: (i, 0)),
            full((EMB, B_VOCAB)),
            full((A_VOCAB, EMB)),
            full((C_VOCAB, EMB)),
            full((EMB, D_INV)),
            full((EMB, D_INV)),
            full((1, D_INV)),
            full((D_INV, D_INV)),
            full((1, D_INV)),
        ],
        out_specs=[
            pl.BlockSpec((N1_BLK, N, EMB, B), lambda i: (i, 0, 0, 0)),
            pl.BlockSpec((INV_BLK, 64, D_INV), lambda i: (i, 0, 0)),
        ],
        out_shape=[
            jax.ShapeDtypeStruct((N, N, EMB, B), jnp.float32),
            jax.ShapeDtypeStruct((B, N, D_INV), jnp.float32),
        ],
    )(btt, atom_types, atom_charges, tbl_t, ta, tc, w1a, w1c, b1r, W2, b2r)

    edges = jnp.transpose(edges_t, (3, 0, 1, 2))
    return invs, edges
